# baseline (speedup 1.0000x reference)
import functools

import jax
import jax.numpy as jnp
from jax import lax
from jax.experimental import pallas as pl
from jax.experimental.pallas import tpu as pltpu
from jax.experimental.pallas import tpu_sc as plsc

BATCH = 16384
EMBED_K = 128
LANES = 16
NUM_CORES = 2
NUM_SUBCORES = 16
NUM_WORKERS = NUM_CORES * NUM_SUBCORES
ROWS_PER_WORKER = BATCH // NUM_WORKERS
CHUNK = 64
NCHUNK = ROWS_PER_WORKER // CHUNK
KCHUNKS = EMBED_K // LANES
GROUPS = CHUNK // LANES
IDX_GROUPS = ROWS_PER_WORKER // LANES

BITREV = (0, 8, 4, 12, 2, 10, 6, 14, 1, 9, 5, 13, 3, 11, 7, 15)


def _make_kernel():
    mesh = plsc.VectorSubcoreMesh(core_axis_name="c", subcore_axis_name="s")

    @functools.partial(
        pl.kernel,
        mesh=mesh,
        out_type=jax.ShapeDtypeStruct((BATCH,), jnp.float32),
        scratch_types=[
            pltpu.VMEM((NCHUNK, CHUNK), jnp.int32),
            pltpu.VMEM((NCHUNK, CHUNK), jnp.int32),
            pltpu.VMEM((2, CHUNK, EMBED_K), jnp.float32),
            pltpu.VMEM((2, CHUNK, EMBED_K), jnp.float32),
            pltpu.VMEM((ROWS_PER_WORKER,), jnp.float32),
            pltpu.VMEM((CHUNK // 2 * LANES,), jnp.float32),
            pltpu.SemaphoreType.DMA,
            pltpu.SemaphoreType.DMA,
            pltpu.SemaphoreType.DMA,
            pltpu.SemaphoreType.DMA,
        ],
    )
    def mf_dot(uidx_hbm, vidx_hbm, w_hbm, h_hbm, out_hbm,
               uidx_v, vidx_v, ubuf, vbuf, outv, pairbuf,
               sem_u0, sem_u1, sem_v0, sem_v1):
        sem_u = (sem_u0, sem_u1)
        sem_v = (sem_v0, sem_v1)
        wid = lax.axis_index("s") * NUM_CORES + lax.axis_index("c")
        base = wid * ROWS_PER_WORKER

        iota = lax.iota(jnp.int32, LANES)
        gdn = lax.GatherDimensionNumbers(
            offset_dims=(), collapsed_slice_dims=(0,), start_index_map=(0,))

        def lane_shuffle(x, idx):
            return lax.gather(
                x, idx[:, None], gdn, slice_sizes=(1,),
                mode=lax.GatherScatterMode.PROMISE_IN_BOUNDS)

        pltpu.sync_copy(uidx_hbm.at[wid], uidx_v)
        pltpu.sync_copy(vidx_hbm.at[wid], vidx_v)

        def start(c):
            slot = c % 2
            cu = pltpu.async_copy(w_hbm.at[uidx_v.at[c]], ubuf.at[slot],
                                  sem_u[slot])
            cv = pltpu.async_copy(h_hbm.at[vidx_v.at[c]], vbuf.at[slot],
                                  sem_v[slot])
            return cu, cv

        def merge(a, b, sh):
            lane = lax.iota(jnp.int32, LANES)
            perm = lane ^ sh
            m = (lane & sh) == 0
            pa = lane_shuffle(a, perm)
            pb = lane_shuffle(b, perm)
            return jnp.where(m, a, pb) + jnp.where(m, pa, b)

        def row_acc(ub, vb, r):
            acc = None
            for i in range(KCHUNKS):
                p = (ub[r, pl.ds(i * LANES, LANES)]
                     * vb[r, pl.ds(i * LANES, LANES)])
                acc = p if acc is None else acc + p
            return acc

        def compute(c):
            slot = c % 2
            ub = ubuf.at[slot]
            vb = vbuf.at[slot]

            @plsc.parallel_loop(0, CHUNK // 2)
            def pair_body(q):
                a = row_acc(ub, vb, 2 * q)
                b = row_acc(ub, vb, 2 * q + 1)
                pairbuf[pl.ds(q * LANES, LANES)] = merge(a, b, 8)

            @plsc.parallel_loop(0, GROUPS)
            def group_body(g):
                lane = lax.iota(jnp.int32, LANES)
                bitrev_v = (((lane & 1) << 3) | ((lane & 2) << 1)
                            | ((lane & 4) >> 1) | ((lane & 8) >> 3))
                vecs = [pairbuf[pl.ds((g * (LANES // 2) + j) * LANES, LANES)]
                        for j in range(LANES // 2)]
                for sh in (4, 2, 1):
                    vecs = [merge(vecs[2 * i], vecs[2 * i + 1], sh)
                            for i in range(len(vecs) // 2)]
                result = lane_shuffle(vecs[0], bitrev_v)
                outv[pl.ds(c * CHUNK + g * LANES, LANES)] = result

        pending = start(0)
        for c in range(NCHUNK):
            nxt = start(c + 1) if c + 1 < NCHUNK else None
            pending[0].wait()
            pending[1].wait()
            compute(c)
            pending = nxt

        pltpu.sync_copy(outv, out_hbm.at[pl.ds(base, ROWS_PER_WORKER)])

    return mf_dot


_mf_dot = _make_kernel()


@jax.jit
def kernel(x, W, H):
    uidx = x[:, 0].astype(jnp.int32).reshape(NUM_WORKERS, NCHUNK, CHUNK)
    vidx = x[:, 1].astype(jnp.int32).reshape(NUM_WORKERS, NCHUNK, CHUNK)
    return _mf_dot(uidx, vidx, W, H)


# speedup vs baseline: 1.0260x; 1.0260x over previous
import functools

import jax
import jax.numpy as jnp
from jax import lax
from jax.experimental import pallas as pl
from jax.experimental.pallas import tpu as pltpu
from jax.experimental.pallas import tpu_sc as plsc

BATCH = 16384
EMBED_K = 128
LANES = 16
NUM_CORES = 2
NUM_SUBCORES = 16
NUM_WORKERS = NUM_CORES * NUM_SUBCORES
ROWS_PER_WORKER = BATCH // NUM_WORKERS
CHUNK = 128
CHUNK_SIZES = (128, 128, 128, 96, 32)
CHUNK_OFFS = (0, 128, 256, 384, 480)
NCHUNK = len(CHUNK_SIZES)
KCHUNKS = EMBED_K // LANES
PAIRS = LANES // 2

BITREV = (0, 8, 4, 12, 2, 10, 6, 14, 1, 9, 5, 13, 3, 11, 7, 15)


def _make_kernel():
    mesh = plsc.VectorSubcoreMesh(core_axis_name="c", subcore_axis_name="s")

    @functools.partial(
        pl.kernel,
        mesh=mesh,
        out_type=jax.ShapeDtypeStruct((BATCH,), jnp.float32),
        scratch_types=[
            pltpu.VMEM((ROWS_PER_WORKER,), jnp.int32),
            pltpu.VMEM((ROWS_PER_WORKER,), jnp.int32),
            pltpu.VMEM((2, CHUNK, EMBED_K), jnp.float32),
            pltpu.VMEM((2, CHUNK, EMBED_K), jnp.float32),
            pltpu.VMEM((ROWS_PER_WORKER,), jnp.float32),
            pltpu.VMEM((CHUNK // 2 * LANES,), jnp.float32),
            pltpu.SemaphoreType.DMA,
            pltpu.SemaphoreType.DMA,
            pltpu.SemaphoreType.DMA,
            pltpu.SemaphoreType.DMA,
        ],
    )
    def mf_dot(uidx_hbm, vidx_hbm, w_hbm, h_hbm, out_hbm,
               uidx_v, vidx_v, ubuf, vbuf, outv, pairbuf,
               sem_u0, sem_u1, sem_v0, sem_v1):
        sem_u = (sem_u0, sem_u1)
        sem_v = (sem_v0, sem_v1)
        wid = lax.axis_index("s") * NUM_CORES + lax.axis_index("c")
        base = wid * ROWS_PER_WORKER

        iota = lax.iota(jnp.int32, LANES)
        gdn = lax.GatherDimensionNumbers(
            offset_dims=(), collapsed_slice_dims=(0,), start_index_map=(0,))

        def lane_shuffle(x, idx):
            return lax.gather(
                x, idx[:, None], gdn, slice_sizes=(1,),
                mode=lax.GatherScatterMode.PROMISE_IN_BOUNDS)

        pltpu.sync_copy(uidx_hbm.at[wid], uidx_v)
        pltpu.sync_copy(vidx_hbm.at[wid], vidx_v)

        def start(c):
            slot = c % 2
            off, size = CHUNK_OFFS[c], CHUNK_SIZES[c]
            cu = pltpu.async_copy(
                w_hbm.at[uidx_v.at[pl.ds(off, size)]],
                ubuf.at[slot, pl.ds(0, size)], sem_u[slot])
            cv = pltpu.async_copy(
                h_hbm.at[vidx_v.at[pl.ds(off, size)]],
                vbuf.at[slot, pl.ds(0, size)], sem_v[slot])
            return cu, cv

        def merge(a, b, sh):
            lane = lax.iota(jnp.int32, LANES)
            perm = lane ^ sh
            m = (lane & sh) == 0
            pa = lane_shuffle(a, perm)
            pb = lane_shuffle(b, perm)
            return jnp.where(m, a, pb) + jnp.where(m, pa, b)

        def row_acc(ub, vb, r):
            acc = None
            for i in range(KCHUNKS):
                p = (ub[r, pl.ds(i * LANES, LANES)]
                     * vb[r, pl.ds(i * LANES, LANES)])
                acc = p if acc is None else acc + p
            return acc

        def compute(c):
            slot = c % 2
            off, size = CHUNK_OFFS[c], CHUNK_SIZES[c]
            ub = ubuf.at[slot]
            vb = vbuf.at[slot]

            @plsc.parallel_loop(0, size // 2)
            def pair_body(q):
                a = row_acc(ub, vb, 2 * q)
                b = row_acc(ub, vb, 2 * q + 1)
                pairbuf[pl.ds(q * LANES, LANES)] = merge(a, b, 8)

            @plsc.parallel_loop(0, size // LANES)
            def group_body(g):
                lane = lax.iota(jnp.int32, LANES)
                bitrev_v = (((lane & 1) << 3) | ((lane & 2) << 1)
                            | ((lane & 4) >> 1) | ((lane & 8) >> 3))
                vecs = [pairbuf[pl.ds((g * PAIRS + j) * LANES, LANES)]
                        for j in range(PAIRS)]
                for sh in (4, 2, 1):
                    vecs = [merge(vecs[2 * i], vecs[2 * i + 1], sh)
                            for i in range(len(vecs) // 2)]
                result = lane_shuffle(vecs[0], bitrev_v)
                outv[pl.ds(off + g * LANES, LANES)] = result

        pending = start(0)
        for c in range(NCHUNK):
            nxt = start(c + 1) if c + 1 < NCHUNK else None
            pending[0].wait()
            pending[1].wait()
            compute(c)
            pending = nxt

        pltpu.sync_copy(outv, out_hbm.at[pl.ds(base, ROWS_PER_WORKER)])

    return mf_dot


_mf_dot = _make_kernel()


@jax.jit
def kernel(x, W, H):
    uidx = x[:, 0].astype(jnp.int32).reshape(NUM_WORKERS, ROWS_PER_WORKER)
    vidx = x[:, 1].astype(jnp.int32).reshape(NUM_WORKERS, ROWS_PER_WORKER)
    return _mf_dot(uidx, vidx, W, H)


# speedup vs baseline: 1.0377x; 1.0114x over previous
import functools

import jax
import jax.numpy as jnp
from jax import lax
from jax.experimental import pallas as pl
from jax.experimental.pallas import tpu as pltpu
from jax.experimental.pallas import tpu_sc as plsc

BATCH = 16384
EMBED_K = 128
LANES = 16
NUM_CORES = 2
NUM_SUBCORES = 16
NUM_WORKERS = NUM_CORES * NUM_SUBCORES
ROWS_PER_WORKER = BATCH // NUM_WORKERS
CHUNK = 128
NCHUNK = ROWS_PER_WORKER // CHUNK
KCHUNKS = EMBED_K // LANES
GROUPS = CHUNK // LANES
IDX_GROUPS = ROWS_PER_WORKER // LANES

BITREV = (0, 8, 4, 12, 2, 10, 6, 14, 1, 9, 5, 13, 3, 11, 7, 15)


def _make_kernel():
    mesh = plsc.VectorSubcoreMesh(core_axis_name="c", subcore_axis_name="s")

    @functools.partial(
        pl.kernel,
        mesh=mesh,
        out_type=jax.ShapeDtypeStruct((BATCH,), jnp.float32),
        scratch_types=[
            pltpu.VMEM((NCHUNK, CHUNK), jnp.int32),
            pltpu.VMEM((NCHUNK, CHUNK), jnp.int32),
            pltpu.VMEM((2, CHUNK, EMBED_K), jnp.float32),
            pltpu.VMEM((2, CHUNK, EMBED_K), jnp.float32),
            pltpu.VMEM((ROWS_PER_WORKER,), jnp.float32),
            pltpu.VMEM((CHUNK // 2 * LANES,), jnp.float32),
            pltpu.SemaphoreType.DMA,
            pltpu.SemaphoreType.DMA,
            pltpu.SemaphoreType.DMA,
            pltpu.SemaphoreType.DMA,
        ],
    )
    def mf_dot(uidx_hbm, vidx_hbm, w_hbm, h_hbm, out_hbm,
               uidx_v, vidx_v, ubuf, vbuf, outv, pairbuf,
               sem_u0, sem_u1, sem_v0, sem_v1):
        sem_u = (sem_u0, sem_u1)
        sem_v = (sem_v0, sem_v1)
        wid = lax.axis_index("s") * NUM_CORES + lax.axis_index("c")
        base = wid * ROWS_PER_WORKER

        iota = lax.iota(jnp.int32, LANES)
        gdn = lax.GatherDimensionNumbers(
            offset_dims=(), collapsed_slice_dims=(0,), start_index_map=(0,))

        def lane_shuffle(x, idx):
            return lax.gather(
                x, idx[:, None], gdn, slice_sizes=(1,),
                mode=lax.GatherScatterMode.PROMISE_IN_BOUNDS)

        pltpu.sync_copy(uidx_hbm.at[wid], uidx_v)
        pltpu.sync_copy(vidx_hbm.at[wid], vidx_v)

        def start(c):
            slot = c % 2
            cu = pltpu.async_copy(w_hbm.at[uidx_v.at[c]], ubuf.at[slot],
                                  sem_u[slot])
            cv = pltpu.async_copy(h_hbm.at[vidx_v.at[c]], vbuf.at[slot],
                                  sem_v[slot])
            return cu, cv

        def merge(a, b, sh):
            lane = lax.iota(jnp.int32, LANES)
            perm = lane ^ sh
            m = (lane & sh) == 0
            pa = lane_shuffle(a, perm)
            pb = lane_shuffle(b, perm)
            return jnp.where(m, a, pb) + jnp.where(m, pa, b)

        def row_acc(ub, vb, r):
            acc = None
            for i in range(KCHUNKS):
                p = (ub[r, pl.ds(i * LANES, LANES)]
                     * vb[r, pl.ds(i * LANES, LANES)])
                acc = p if acc is None else acc + p
            return acc

        def compute(c):
            slot = c % 2
            ub = ubuf.at[slot]
            vb = vbuf.at[slot]

            @plsc.parallel_loop(0, CHUNK // 2)
            def pair_body(q):
                a = row_acc(ub, vb, 2 * q)
                b = row_acc(ub, vb, 2 * q + 1)
                pairbuf[pl.ds(q * LANES, LANES)] = merge(a, b, 8)

            @plsc.parallel_loop(0, GROUPS)
            def group_body(g):
                lane = lax.iota(jnp.int32, LANES)
                bitrev_v = (((lane & 1) << 3) | ((lane & 2) << 1)
                            | ((lane & 4) >> 1) | ((lane & 8) >> 3))
                vecs = [pairbuf[pl.ds((g * GROUPS + j) * LANES, LANES)]
                        for j in range(GROUPS)]
                for sh in (4, 2, 1):
                    vecs = [merge(vecs[2 * i], vecs[2 * i + 1], sh)
                            for i in range(len(vecs) // 2)]
                result = lane_shuffle(vecs[0], bitrev_v)
                outv[pl.ds(c * CHUNK + g * LANES, LANES)] = result

        pending = start(0)
        for c in range(NCHUNK):
            nxt = start(c + 1) if c + 1 < NCHUNK else None
            pending[0].wait()
            pending[1].wait()
            compute(c)
            pending = nxt

        pltpu.sync_copy(outv, out_hbm.at[pl.ds(base, ROWS_PER_WORKER)])

    return mf_dot


_mf_dot = _make_kernel()


@jax.jit
def kernel(x, W, H):
    uidx = x[:, 0].astype(jnp.int32).reshape(NUM_WORKERS, NCHUNK, CHUNK)
    vidx = x[:, 1].astype(jnp.int32).reshape(NUM_WORKERS, NCHUNK, CHUNK)
    return _mf_dot(uidx, vidx, W, H)


# speedup vs baseline: 1.0565x; 1.0181x over previous
import functools

import jax
import jax.numpy as jnp
from jax import lax
from jax.experimental import pallas as pl
from jax.experimental.pallas import tpu as pltpu
from jax.experimental.pallas import tpu_sc as plsc

BATCH = 16384
EMBED_K = 128
LANES = 16
NUM_CORES = 2
NUM_SUBCORES = 16
NUM_WORKERS = NUM_CORES * NUM_SUBCORES
ROWS_PER_WORKER = BATCH // NUM_WORKERS
CHUNK = 128
NCHUNK = ROWS_PER_WORKER // CHUNK
KCHUNKS = EMBED_K // LANES
GROUPS = CHUNK // LANES
IDX_GROUPS = ROWS_PER_WORKER // LANES

BITREV = (0, 8, 4, 12, 2, 10, 6, 14, 1, 9, 5, 13, 3, 11, 7, 15)


def _make_kernel():
    mesh = plsc.VectorSubcoreMesh(core_axis_name="c", subcore_axis_name="s")

    @functools.partial(
        pl.kernel,
        mesh=mesh,
        out_type=jax.ShapeDtypeStruct((BATCH,), jnp.float32),
        scratch_types=[
            pltpu.VMEM((NCHUNK, CHUNK), jnp.int32),
            pltpu.VMEM((NCHUNK, CHUNK), jnp.int32),
            pltpu.VMEM((2, CHUNK, EMBED_K), jnp.float32),
            pltpu.VMEM((2, CHUNK, EMBED_K), jnp.float32),
            pltpu.VMEM((ROWS_PER_WORKER,), jnp.float32),
            pltpu.VMEM((CHUNK // 2 * LANES,), jnp.float32),
            pltpu.SemaphoreType.DMA,
            pltpu.SemaphoreType.DMA,
            pltpu.SemaphoreType.DMA,
            pltpu.SemaphoreType.DMA,
            pltpu.SemaphoreType.DMA,
            pltpu.SemaphoreType.DMA,
        ],
    )
    def mf_dot(uidx_hbm, vidx_hbm, w_hbm, h_hbm, out_hbm,
               uidx_v, vidx_v, ubuf, vbuf, outv, pairbuf,
               sem_u0, sem_u1, sem_v0, sem_v1, sem_idx, sem_out):
        sem_u = (sem_u0, sem_u1)
        sem_v = (sem_v0, sem_v1)
        wid = lax.axis_index("s") * NUM_CORES + lax.axis_index("c")
        base = wid * ROWS_PER_WORKER

        iota = lax.iota(jnp.int32, LANES)
        gdn = lax.GatherDimensionNumbers(
            offset_dims=(), collapsed_slice_dims=(0,), start_index_map=(0,))

        def lane_shuffle(x, idx):
            return lax.gather(
                x, idx[:, None], gdn, slice_sizes=(1,),
                mode=lax.GatherScatterMode.PROMISE_IN_BOUNDS)

        cpu = pltpu.async_copy(uidx_hbm.at[wid], uidx_v, sem_idx)
        cpv = pltpu.async_copy(vidx_hbm.at[wid], vidx_v, sem_out)
        cpu.wait()
        cpv.wait()

        def start(c):
            slot = c % 2
            cu = pltpu.async_copy(w_hbm.at[uidx_v.at[c]], ubuf.at[slot],
                                  sem_u[slot])
            cv = pltpu.async_copy(h_hbm.at[vidx_v.at[c]], vbuf.at[slot],
                                  sem_v[slot])
            return cu, cv

        def merge(a, b, sh):
            lane = lax.iota(jnp.int32, LANES)
            perm = lane ^ sh
            m = (lane & sh) == 0
            pa = lane_shuffle(a, perm)
            pb = lane_shuffle(b, perm)
            return jnp.where(m, a, pb) + jnp.where(m, pa, b)

        def row_acc(ub, vb, r):
            acc = None
            for i in range(KCHUNKS):
                p = (ub[r, pl.ds(i * LANES, LANES)]
                     * vb[r, pl.ds(i * LANES, LANES)])
                acc = p if acc is None else acc + p
            return acc

        def compute(c):
            slot = c % 2
            ub = ubuf.at[slot]
            vb = vbuf.at[slot]

            @plsc.parallel_loop(0, CHUNK // 2)
            def pair_body(q):
                a = row_acc(ub, vb, 2 * q)
                b = row_acc(ub, vb, 2 * q + 1)
                pairbuf[pl.ds(q * LANES, LANES)] = merge(a, b, 8)

            @plsc.parallel_loop(0, GROUPS)
            def group_body(g):
                lane = lax.iota(jnp.int32, LANES)
                bitrev_v = (((lane & 1) << 3) | ((lane & 2) << 1)
                            | ((lane & 4) >> 1) | ((lane & 8) >> 3))
                vecs = [pairbuf[pl.ds((g * GROUPS + j) * LANES, LANES)]
                        for j in range(GROUPS)]
                for sh in (4, 2, 1):
                    vecs = [merge(vecs[2 * i], vecs[2 * i + 1], sh)
                            for i in range(len(vecs) // 2)]
                result = lane_shuffle(vecs[0], bitrev_v)
                outv[pl.ds(c * CHUNK + g * LANES, LANES)] = result

        pending = start(0)
        outcps = []
        for c in range(NCHUNK):
            nxt = start(c + 1) if c + 1 < NCHUNK else None
            pending[0].wait()
            pending[1].wait()
            compute(c)
            outcps.append(pltpu.async_copy(
                outv.at[pl.ds(c * CHUNK, CHUNK)],
                out_hbm.at[pl.ds(base + c * CHUNK, CHUNK)], sem_out))
            pending = nxt

        for cp in outcps:
            cp.wait()

    return mf_dot


_mf_dot = _make_kernel()


@jax.jit
def kernel(x, W, H):
    uidx = x[:, 0].astype(jnp.int32).reshape(NUM_WORKERS, NCHUNK, CHUNK)
    vidx = x[:, 1].astype(jnp.int32).reshape(NUM_WORKERS, NCHUNK, CHUNK)
    return _mf_dot(uidx, vidx, W, H)


# speedup vs baseline: 1.0763x; 1.0187x over previous
import functools

import jax
import jax.numpy as jnp
from jax import lax
from jax.experimental import pallas as pl
from jax.experimental.pallas import tpu as pltpu
from jax.experimental.pallas import tpu_sc as plsc

BATCH = 16384
EMBED_K = 128
LANES = 16
NUM_CORES = 2
NUM_SUBCORES = 16
NUM_WORKERS = NUM_CORES * NUM_SUBCORES
ROWS_PER_WORKER = BATCH // NUM_WORKERS
CHUNK = 128
NCHUNK = ROWS_PER_WORKER // CHUNK
KCHUNKS = EMBED_K // LANES
GROUPS = CHUNK // LANES
IDX_GROUPS = ROWS_PER_WORKER // LANES

BITREV = (0, 8, 4, 12, 2, 10, 6, 14, 1, 9, 5, 13, 3, 11, 7, 15)


def _make_kernel():
    mesh = plsc.VectorSubcoreMesh(core_axis_name="c", subcore_axis_name="s")

    @functools.partial(
        pl.kernel,
        mesh=mesh,
        out_type=jax.ShapeDtypeStruct((BATCH,), jnp.float32),
        scratch_types=[
            pltpu.VMEM((NCHUNK, CHUNK), jnp.int32),
            pltpu.VMEM((NCHUNK, CHUNK), jnp.int32),
            pltpu.VMEM((3, CHUNK, EMBED_K), jnp.float32),
            pltpu.VMEM((3, CHUNK, EMBED_K), jnp.float32),
            pltpu.VMEM((ROWS_PER_WORKER,), jnp.float32),
            pltpu.VMEM((CHUNK // 2 * LANES,), jnp.float32),
            pltpu.SemaphoreType.DMA,
            pltpu.SemaphoreType.DMA,
            pltpu.SemaphoreType.DMA,
            pltpu.SemaphoreType.DMA,
            pltpu.SemaphoreType.DMA,
            pltpu.SemaphoreType.DMA,
            pltpu.SemaphoreType.DMA,
            pltpu.SemaphoreType.DMA,
        ],
    )
    def mf_dot(uidx_hbm, vidx_hbm, w_hbm, h_hbm, out_hbm,
               uidx_v, vidx_v, ubuf, vbuf, outv, pairbuf,
               sem_u0, sem_u1, sem_u2, sem_v0, sem_v1, sem_v2,
               sem_idx, sem_out):
        sem_u = (sem_u0, sem_u1, sem_u2)
        sem_v = (sem_v0, sem_v1, sem_v2)
        wid = lax.axis_index("s") * NUM_CORES + lax.axis_index("c")
        base = wid * ROWS_PER_WORKER

        iota = lax.iota(jnp.int32, LANES)
        gdn = lax.GatherDimensionNumbers(
            offset_dims=(), collapsed_slice_dims=(0,), start_index_map=(0,))

        def lane_shuffle(x, idx):
            return lax.gather(
                x, idx[:, None], gdn, slice_sizes=(1,),
                mode=lax.GatherScatterMode.PROMISE_IN_BOUNDS)

        cpu = pltpu.async_copy(uidx_hbm.at[wid], uidx_v, sem_idx)
        cpv = pltpu.async_copy(vidx_hbm.at[wid], vidx_v, sem_out)
        cpu.wait()
        cpv.wait()

        def start(c):
            slot = c % 3
            cu = pltpu.async_copy(w_hbm.at[uidx_v.at[c]], ubuf.at[slot],
                                  sem_u[slot])
            cv = pltpu.async_copy(h_hbm.at[vidx_v.at[c]], vbuf.at[slot],
                                  sem_v[slot])
            return cu, cv

        def merge(a, b, sh):
            lane = lax.iota(jnp.int32, LANES)
            perm = lane ^ sh
            m = (lane & sh) == 0
            pa = lane_shuffle(a, perm)
            pb = lane_shuffle(b, perm)
            return jnp.where(m, a, pb) + jnp.where(m, pa, b)

        def row_acc(ub, vb, r):
            acc = None
            for i in range(KCHUNKS):
                p = (ub[r, pl.ds(i * LANES, LANES)]
                     * vb[r, pl.ds(i * LANES, LANES)])
                acc = p if acc is None else acc + p
            return acc

        def compute(c):
            slot = c % 3
            ub = ubuf.at[slot]
            vb = vbuf.at[slot]

            @plsc.parallel_loop(0, CHUNK // 2)
            def pair_body(q):
                a = row_acc(ub, vb, 2 * q)
                b = row_acc(ub, vb, 2 * q + 1)
                pairbuf[pl.ds(q * LANES, LANES)] = merge(a, b, 8)

            @plsc.parallel_loop(0, GROUPS)
            def group_body(g):
                lane = lax.iota(jnp.int32, LANES)
                bitrev_v = (((lane & 1) << 3) | ((lane & 2) << 1)
                            | ((lane & 4) >> 1) | ((lane & 8) >> 3))
                vecs = [pairbuf[pl.ds((g * GROUPS + j) * LANES, LANES)]
                        for j in range(GROUPS)]
                for sh in (4, 2, 1):
                    vecs = [merge(vecs[2 * i], vecs[2 * i + 1], sh)
                            for i in range(len(vecs) // 2)]
                result = lane_shuffle(vecs[0], bitrev_v)
                outv[pl.ds(c * CHUNK + g * LANES, LANES)] = result

        inflight = [start(0), start(1)]
        outcps = []
        for c in range(NCHUNK):
            if c + 2 < NCHUNK:
                inflight.append(start(c + 2))
            pending = inflight.pop(0)
            pending[0].wait()
            pending[1].wait()
            compute(c)
            outcps.append(pltpu.async_copy(
                outv.at[pl.ds(c * CHUNK, CHUNK)],
                out_hbm.at[pl.ds(base + c * CHUNK, CHUNK)], sem_out))

        for cp in outcps:
            cp.wait()

    return mf_dot


_mf_dot = _make_kernel()


@jax.jit
def kernel(x, W, H):
    uidx = x[:, 0].astype(jnp.int32).reshape(NUM_WORKERS, NCHUNK, CHUNK)
    vidx = x[:, 1].astype(jnp.int32).reshape(NUM_WORKERS, NCHUNK, CHUNK)
    return _mf_dot(uidx, vidx, W, H)


# speedup vs baseline: 1.0825x; 1.0057x over previous
import functools

import jax
import jax.numpy as jnp
from jax import lax
from jax.experimental import pallas as pl
from jax.experimental.pallas import tpu as pltpu
from jax.experimental.pallas import tpu_sc as plsc

BATCH = 16384
EMBED_K = 128
LANES = 16
NUM_CORES = 2
NUM_SUBCORES = 16
NUM_WORKERS = NUM_CORES * NUM_SUBCORES
ROWS_PER_WORKER = BATCH // NUM_WORKERS
CHUNK = 64
NCHUNK = ROWS_PER_WORKER // CHUNK
KCHUNKS = EMBED_K // LANES
GROUPS = CHUNK // LANES
IDX_GROUPS = ROWS_PER_WORKER // LANES

BITREV = (0, 8, 4, 12, 2, 10, 6, 14, 1, 9, 5, 13, 3, 11, 7, 15)


def _make_kernel():
    mesh = plsc.VectorSubcoreMesh(core_axis_name="c", subcore_axis_name="s")

    @functools.partial(
        pl.kernel,
        mesh=mesh,
        out_type=jax.ShapeDtypeStruct((BATCH,), jnp.float32),
        scratch_types=[
            pltpu.VMEM((NCHUNK, CHUNK), jnp.int32),
            pltpu.VMEM((NCHUNK, CHUNK), jnp.int32),
            pltpu.VMEM((6, CHUNK, EMBED_K), jnp.float32),
            pltpu.VMEM((6, CHUNK, EMBED_K), jnp.float32),
            pltpu.VMEM((ROWS_PER_WORKER,), jnp.float32),
            pltpu.VMEM((CHUNK // 2 * LANES,), jnp.float32),
            pltpu.SemaphoreType.DMA,
            pltpu.SemaphoreType.DMA,
            pltpu.SemaphoreType.DMA,
            pltpu.SemaphoreType.DMA,
            pltpu.SemaphoreType.DMA,
            pltpu.SemaphoreType.DMA,
            pltpu.SemaphoreType.DMA,
            pltpu.SemaphoreType.DMA,
            pltpu.SemaphoreType.DMA,
            pltpu.SemaphoreType.DMA,
            pltpu.SemaphoreType.DMA,
            pltpu.SemaphoreType.DMA,
            pltpu.SemaphoreType.DMA,
            pltpu.SemaphoreType.DMA,
        ],
    )
    def mf_dot(uidx_hbm, vidx_hbm, w_hbm, h_hbm, out_hbm,
               uidx_v, vidx_v, ubuf, vbuf, outv, pairbuf,
               sem_u0, sem_u1, sem_u2, sem_u3, sem_u4, sem_u5,
               sem_v0, sem_v1, sem_v2, sem_v3, sem_v4, sem_v5,
               sem_idx, sem_out):
        sem_u = (sem_u0, sem_u1, sem_u2, sem_u3, sem_u4, sem_u5)
        sem_v = (sem_v0, sem_v1, sem_v2, sem_v3, sem_v4, sem_v5)
        wid = lax.axis_index("s") * NUM_CORES + lax.axis_index("c")
        base = wid * ROWS_PER_WORKER

        iota = lax.iota(jnp.int32, LANES)
        gdn = lax.GatherDimensionNumbers(
            offset_dims=(), collapsed_slice_dims=(0,), start_index_map=(0,))

        def lane_shuffle(x, idx):
            return lax.gather(
                x, idx[:, None], gdn, slice_sizes=(1,),
                mode=lax.GatherScatterMode.PROMISE_IN_BOUNDS)

        cpu = pltpu.async_copy(uidx_hbm.at[wid], uidx_v, sem_idx)
        cpv = pltpu.async_copy(vidx_hbm.at[wid], vidx_v, sem_out)
        cpu.wait()
        cpv.wait()

        def start(c):
            slot = c % 6
            cu = pltpu.async_copy(w_hbm.at[uidx_v.at[c]], ubuf.at[slot],
                                  sem_u[slot])
            cv = pltpu.async_copy(h_hbm.at[vidx_v.at[c]], vbuf.at[slot],
                                  sem_v[slot])
            return cu, cv

        def merge(a, b, sh):
            lane = lax.iota(jnp.int32, LANES)
            perm = lane ^ sh
            m = (lane & sh) == 0
            pa = lane_shuffle(a, perm)
            pb = lane_shuffle(b, perm)
            return jnp.where(m, a, pb) + jnp.where(m, pa, b)

        def row_acc(ub, vb, r):
            acc = None
            for i in range(KCHUNKS):
                p = (ub[r, pl.ds(i * LANES, LANES)]
                     * vb[r, pl.ds(i * LANES, LANES)])
                acc = p if acc is None else acc + p
            return acc

        def compute(c):
            slot = c % 6
            ub = ubuf.at[slot]
            vb = vbuf.at[slot]

            @plsc.parallel_loop(0, CHUNK // 2)
            def pair_body(q):
                a = row_acc(ub, vb, 2 * q)
                b = row_acc(ub, vb, 2 * q + 1)
                pairbuf[pl.ds(q * LANES, LANES)] = merge(a, b, 8)

            @plsc.parallel_loop(0, GROUPS)
            def group_body(g):
                lane = lax.iota(jnp.int32, LANES)
                bitrev_v = (((lane & 1) << 3) | ((lane & 2) << 1)
                            | ((lane & 4) >> 1) | ((lane & 8) >> 3))
                vecs = [pairbuf[pl.ds((g * (LANES // 2) + j) * LANES, LANES)]
                        for j in range(LANES // 2)]
                for sh in (4, 2, 1):
                    vecs = [merge(vecs[2 * i], vecs[2 * i + 1], sh)
                            for i in range(len(vecs) // 2)]
                result = lane_shuffle(vecs[0], bitrev_v)
                outv[pl.ds(c * CHUNK + g * LANES, LANES)] = result

        inflight = [start(0), start(1), start(2), start(3)]
        outcps = []
        for c in range(NCHUNK):
            if c + 4 < NCHUNK:
                inflight.append(start(c + 4))
            pending = inflight.pop(0)
            pending[0].wait()
            pending[1].wait()
            compute(c)
            outcps.append(pltpu.async_copy(
                outv.at[pl.ds(c * CHUNK, CHUNK)],
                out_hbm.at[pl.ds(base + c * CHUNK, CHUNK)], sem_out))

        for cp in outcps:
            cp.wait()

    return mf_dot


_mf_dot = _make_kernel()


@jax.jit
def kernel(x, W, H):
    uidx = x[:, 0].astype(jnp.int32).reshape(NUM_WORKERS, NCHUNK, CHUNK)
    vidx = x[:, 1].astype(jnp.int32).reshape(NUM_WORKERS, NCHUNK, CHUNK)
    return _mf_dot(uidx, vidx, W, H)
